# baseline (speedup 1.0000x reference)
import functools

import jax
import jax.numpy as jnp
from jax import lax
from jax.experimental import pallas as pl
from jax.experimental.pallas import tpu as pltpu
from jax.experimental.pallas import tpu_sc as plsc

B = 4096
L = 200
D = 64
N = B * L
NW = 32
PER_W = N // NW
TPC = 20
CHUNK = 16 * TPC
CPT = (L // 2) // TPC
NCHUNK = PER_W // CHUNK
NBUF = 4


def _gather_body(idx_hbm, table_hbm, out_hbm, idx_raw, pidx, rows_v,
                 gsem, wsem):
    wid = lax.axis_index("s") * 2 + lax.axis_index("c")
    base = wid * PER_W

    pltpu.sync_copy(idx_hbm.at[wid], idx_raw)

    lane = lax.iota(jnp.int32, 16)
    v0 = (lane >> 1) * L + (lane & 1)

    def build(c, b):
        ii = c // CPT
        j0 = (c % CPT) * TPC
        off = (8 * L) * ii + 2 * j0
        dst = pidx.at[b]
        for q in range(TPC):
            vals = plsc.load_gather(idx_raw, [v0 + (off + 2 * q)])
            dst[pl.ds(q * 16, 16)] = vals

    def gather(b):
        return pltpu.make_async_copy(
            table_hbm.at[pidx.at[b]], rows_v.at[b], gsem.at[b])

    def write(i, b):
        return pltpu.make_async_copy(
            rows_v.at[b], out_hbm.at[pl.ds(base + i * CHUNK, CHUNK)],
            wsem.at[b])

    for b in range(NBUF):
        build(b, b)
        gather(b).start()

    def group(g, carry):
        for b in range(NBUF):
            i = g + b
            gather(b).wait()
            write(i, b).start()
        for b in range(NBUF):
            i = g + b
            nxt = i + NBUF

            @pl.when(nxt < NCHUNK)
            def _():
                write(i, b).wait()
                build(nxt, b)
                gather(b).start()

        return carry

    lax.fori_loop(0, NCHUNK // NBUF, lambda k, c: group(k * NBUF, c), 0)

    for b in range(NBUF):
        write(NCHUNK - NBUF + b, b).wait()


V = 1000000
NFB = V // 128
TAIL = V - NFB * 128


BPW = NFB // NW


def _transpose_body(tt_hbm, tail_hbm, out_hbm, blk0, blk1, tcol0, tcol1,
                    tmem, gsem0, gsem1, wsem0, wsem1):
    wid = lax.axis_index("s") * 2 + lax.axis_index("c")
    start = BPW * wid

    lane = lax.iota(jnp.int32, 16)

    def rd(t, blk_b, gsem_b):
        return pltpu.make_async_copy(
            tt_hbm.at[:, pl.ds(t * 128, 128)], blk_b, gsem_b)

    def wr(t, tcol_b, wsem_b):
        return pltpu.make_async_copy(
            tcol_b, out_hbm.at[pl.ds(t * 8192, 8192)], wsem_b)

    def txp(src, dst, npair):
        for u in range(npair):
            for h in range(2):
                c = jnp.full((16,), 2 * u + h, jnp.int32)
                for q in range(4):
                    vals = plsc.load_gather(src, [lane + 16 * q, c])
                    dst[pl.ds(u * 128 + h * 64 + q * 16, 16)] = vals

    bufs = [(blk0, tcol0, gsem0, wsem0), (blk1, tcol1, gsem1, wsem1)]

    rd(start, blk0, gsem0).start()
    rd(start + 1, blk1, gsem1).start()

    def group(g, carry):
        for b in range(2):
            k = 2 * g + b
            t = start + k
            blk_b, tcol_b, gsem_b, wsem_b = bufs[b]
            rd(t, blk_b, gsem_b).wait()

            @pl.when(k >= 2)
            def _():
                wr(t - 2, tcol_b, wsem_b).wait()

            txp(blk_b, tcol_b, 64)
            wr(t, tcol_b, wsem_b).start()

            @pl.when(k + 2 < BPW)
            def _():
                rd(t + 2, blk_b, gsem_b).start()

        return carry

    lax.fori_loop(0, BPW // 2, group, 0)
    wr(start + BPW - 2, tcol0, wsem0).wait()
    wr(start + BPW - 1, tcol1, wsem1).wait()

    @pl.when(wid < NFB - NW * BPW)
    def _():
        t = NW * BPW + wid
        pltpu.sync_copy(tt_hbm.at[:, pl.ds(t * 128, 128)], blk0)
        txp(blk0, tcol0, 64)
        pltpu.sync_copy(tcol0, out_hbm.at[pl.ds(t * 8192, 8192)])

    @pl.when(wid == NW - 1)
    def _():
        pltpu.sync_copy(tail_hbm, tmem)
        for r in range(TAIL // 2):
            for q in range(8):
                tcol1[pl.ds(r * 128 + 16 * q, 16)] = (
                    tmem[r, pl.ds(16 * q, 16)])
        pltpu.sync_copy(tcol1.at[pl.ds(0, TAIL * 64)],
                        out_hbm.at[pl.ds(NFB * 8192, TAIL * 64)])


_transpose = functools.partial(
    pl.kernel,
    out_type=jax.ShapeDtypeStruct((V * D,), jnp.float32),
    mesh=plsc.VectorSubcoreMesh(core_axis_name="c", subcore_axis_name="s"),
    scratch_types=[
        pltpu.VMEM((64, 128), jnp.float32),
        pltpu.VMEM((64, 128), jnp.float32),
        pltpu.VMEM((8192,), jnp.float32),
        pltpu.VMEM((8192,), jnp.float32),
        pltpu.VMEM((TAIL // 2, 128), jnp.float32),
        pltpu.SemaphoreType.DMA,
        pltpu.SemaphoreType.DMA,
        pltpu.SemaphoreType.DMA,
        pltpu.SemaphoreType.DMA,
    ],
    compiler_params=pltpu.CompilerParams(
        use_tc_tiling_on_sc=True, needs_layout_passes=False),
)(_transpose_body)


_gather = functools.partial(
    pl.kernel,
    out_type=jax.ShapeDtypeStruct((N, D), jnp.float32),
    mesh=plsc.VectorSubcoreMesh(core_axis_name="c", subcore_axis_name="s"),
    scratch_types=[
        pltpu.VMEM((PER_W,), jnp.int32),
        pltpu.VMEM((NBUF, CHUNK), jnp.int32),
        pltpu.VMEM((NBUF, CHUNK, D), jnp.float32),
        pltpu.SemaphoreType.DMA((NBUF,)),
        pltpu.SemaphoreType.DMA((NBUF,)),
    ],
    compiler_params=pltpu.CompilerParams(
        use_tc_tiling_on_sc=False, needs_layout_passes=False),
)(_gather_body)


@jax.jit
def kernel(inputs, table):
    idx = inputs.reshape(NW, PER_W)
    tail = table[NFB * 128:].reshape(TAIL // 2, 2 * D)
    tab = _transpose(table.T, tail).reshape(V, D)
    rows = _gather(idx, tab)
    out4 = rows.reshape(B // 8, L // 2, 8, 2 * D)
    return out4.transpose(0, 2, 1, 3).reshape(B, L * D)


# speedup vs baseline: 1.2793x; 1.2793x over previous
import functools

import jax
import jax.numpy as jnp
from jax import lax
from jax.experimental import pallas as pl
from jax.experimental.pallas import tpu as pltpu
from jax.experimental.pallas import tpu_sc as plsc

B = 4096
L = 200
D = 64
N = B * L
NW = 32
PER_W = N // NW
TPC = 20
CHUNK = 16 * TPC
CPT = (L // 2) // TPC
NCHUNK = PER_W // CHUNK
NBUF = 4


def _gather_body(idx_hbm, table_hbm, out_hbm, idx_raw, pidx, rows_v,
                 gsem, wsem):
    wid = lax.axis_index("s") * 2 + lax.axis_index("c")
    base = wid * PER_W

    pltpu.sync_copy(idx_hbm.at[wid], idx_raw)

    lane = lax.iota(jnp.int32, 16)
    v0 = (lane >> 1) * L + (lane & 1)

    def build(c, b):
        ii = c // CPT
        j0 = (c % CPT) * TPC
        off = (8 * L) * ii + 2 * j0
        dst = pidx.at[b]
        for q in range(TPC):
            vals = plsc.load_gather(idx_raw, [v0 + (off + 2 * q)])
            dst[pl.ds(q * 16, 16)] = vals

    def gather(b):
        return pltpu.make_async_copy(
            table_hbm.at[pidx.at[b]], rows_v.at[b], gsem.at[b])

    def write(i, b):
        return pltpu.make_async_copy(
            rows_v.at[b], out_hbm.at[pl.ds(base + i * CHUNK, CHUNK)],
            wsem.at[b])

    for b in range(NBUF):
        build(b, b)
        gather(b).start()

    def group(g, carry):
        for b in range(NBUF):
            i = g + b
            gather(b).wait()
            write(i, b).start()
        for b in range(NBUF):
            i = g + b
            nxt = i + NBUF

            @pl.when(nxt < NCHUNK)
            def _():
                write(i, b).wait()
                build(nxt, b)
                gather(b).start()

        return carry

    lax.fori_loop(0, NCHUNK // NBUF, lambda k, c: group(k * NBUF, c), 0)

    for b in range(NBUF):
        write(NCHUNK - NBUF + b, b).wait()


V = 1000000
NFB = V // 128
TAIL = V - NFB * 128


BPW = NFB // NW


def _transpose_body(tt_hbm, tail_hbm, out_hbm, blk0, blk1, tcol0, tcol1,
                    tmem, gsem0, gsem1, wsem0, wsem1):
    wid = lax.axis_index("s") * 2 + lax.axis_index("c")
    start = BPW * wid

    lane = lax.iota(jnp.int32, 16)

    def rd(t, blk_b, gsem_b):
        return pltpu.make_async_copy(
            tt_hbm.at[:, pl.ds(t * 128, 128)], blk_b, gsem_b)

    def wr(t, tcol_b, wsem_b):
        return pltpu.make_async_copy(
            tcol_b, out_hbm.at[pl.ds(t * 8192, 8192)], wsem_b)

    sidx = [lane * 64 + r for r in range(8)]

    def txp(src, dst):
        def loads(d):
            return [src[d, pl.ds(16 * q, 16)] for q in range(8)]

        def stores(d, vals):
            for q in range(8):
                plsc.store_scatter(
                    dst.at[pl.ds(1024 * q + 8 * (d // 8), 968)],
                    [sidx[d % 8]], vals[q])

        prev = loads(0)
        for d in range(1, 64):
            cur = loads(d)
            stores(d - 1, prev)
            prev = cur
        stores(63, prev)

    bufs = [(blk0, tcol0, gsem0, wsem0), (blk1, tcol1, gsem1, wsem1)]

    rd(start, blk0, gsem0).start()
    rd(start + 1, blk1, gsem1).start()

    def group(g, carry):
        for b in range(2):
            k = 2 * g + b
            t = start + k
            blk_b, tcol_b, gsem_b, wsem_b = bufs[b]
            rd(t, blk_b, gsem_b).wait()

            @pl.when(k >= 2)
            def _():
                wr(t - 2, tcol_b, wsem_b).wait()

            txp(blk_b, tcol_b)
            wr(t, tcol_b, wsem_b).start()

            @pl.when(k + 2 < BPW)
            def _():
                rd(t + 2, blk_b, gsem_b).start()

        return carry

    lax.fori_loop(0, BPW // 2, group, 0)
    wr(start + BPW - 2, tcol0, wsem0).wait()
    wr(start + BPW - 1, tcol1, wsem1).wait()

    @pl.when(wid < NFB - NW * BPW)
    def _():
        t = NW * BPW + wid
        pltpu.sync_copy(tt_hbm.at[:, pl.ds(t * 128, 128)], blk0)
        txp(blk0, tcol0)
        pltpu.sync_copy(tcol0, out_hbm.at[pl.ds(t * 8192, 8192)])

    @pl.when(wid == NW - 1)
    def _():
        pltpu.sync_copy(tail_hbm, tmem)
        for r in range(TAIL // 2):
            for q in range(8):
                tcol1[pl.ds(r * 128 + 16 * q, 16)] = (
                    tmem[r, pl.ds(16 * q, 16)])
        pltpu.sync_copy(tcol1.at[pl.ds(0, TAIL * 64)],
                        out_hbm.at[pl.ds(NFB * 8192, TAIL * 64)])


_transpose = functools.partial(
    pl.kernel,
    out_type=jax.ShapeDtypeStruct((V * D,), jnp.float32),
    mesh=plsc.VectorSubcoreMesh(core_axis_name="c", subcore_axis_name="s"),
    scratch_types=[
        pltpu.VMEM((64, 128), jnp.float32),
        pltpu.VMEM((64, 128), jnp.float32),
        pltpu.VMEM((8192,), jnp.float32),
        pltpu.VMEM((8192,), jnp.float32),
        pltpu.VMEM((TAIL // 2, 128), jnp.float32),
        pltpu.SemaphoreType.DMA,
        pltpu.SemaphoreType.DMA,
        pltpu.SemaphoreType.DMA,
        pltpu.SemaphoreType.DMA,
    ],
    compiler_params=pltpu.CompilerParams(
        use_tc_tiling_on_sc=True, needs_layout_passes=False),
)(_transpose_body)


_gather = functools.partial(
    pl.kernel,
    out_type=jax.ShapeDtypeStruct((N, D), jnp.float32),
    mesh=plsc.VectorSubcoreMesh(core_axis_name="c", subcore_axis_name="s"),
    scratch_types=[
        pltpu.VMEM((PER_W,), jnp.int32),
        pltpu.VMEM((NBUF, CHUNK), jnp.int32),
        pltpu.VMEM((NBUF, CHUNK, D), jnp.float32),
        pltpu.SemaphoreType.DMA((NBUF,)),
        pltpu.SemaphoreType.DMA((NBUF,)),
    ],
    compiler_params=pltpu.CompilerParams(
        use_tc_tiling_on_sc=False, needs_layout_passes=False),
)(_gather_body)


@jax.jit
def kernel(inputs, table):
    idx = inputs.reshape(NW, PER_W)
    tail = table[NFB * 128:].reshape(TAIL // 2, 2 * D)
    tab = _transpose(table.T, tail).reshape(V, D)
    rows = _gather(idx, tab)
    out4 = rows.reshape(B // 8, L // 2, 8, 2 * D)
    return out4.transpose(0, 2, 1, 3).reshape(B, L * D)


# speedup vs baseline: 2.2963x; 1.7950x over previous
import functools

import jax
import jax.numpy as jnp
from jax import lax
from jax.experimental import pallas as pl
from jax.experimental.pallas import tpu as pltpu
from jax.experimental.pallas import tpu_sc as plsc

B = 4096
L = 200
D = 64
N = B * L
NW = 32
PER_W = N // NW
TPC = 20
CHUNK = 16 * TPC
CPT = (L // 2) // TPC
NCHUNK = PER_W // CHUNK
NBUF = 4


def _gather_body(idx_hbm, table_hbm, out_hbm, idx_raw, pidx, rows_v,
                 gsem, wsem):
    wid = lax.axis_index("s") * 2 + lax.axis_index("c")
    base = wid * PER_W

    pltpu.sync_copy(idx_hbm.at[wid], idx_raw)

    lane = lax.iota(jnp.int32, 16)
    v0 = (lane >> 1) * L + (lane & 1)

    def build(c, b):
        ii = c // CPT
        j0 = (c % CPT) * TPC
        off = (8 * L) * ii + 2 * j0
        dst = pidx.at[b]
        for q in range(TPC):
            vals = plsc.load_gather(idx_raw, [v0 + (off + 2 * q)])
            dst[pl.ds(q * 16, 16)] = vals

    def gather(b):
        return pltpu.make_async_copy(
            table_hbm.at[pidx.at[b]], rows_v.at[b], gsem.at[b])

    def write(i, b):
        return pltpu.make_async_copy(
            rows_v.at[b], out_hbm.at[pl.ds(base + i * CHUNK, CHUNK)],
            wsem.at[b])

    for b in range(NBUF):
        build(b, b)
        gather(b).start()

    def group(g, carry):
        for b in range(NBUF):
            i = g + b
            gather(b).wait()
            write(i, b).start()
        for b in range(NBUF):
            i = g + b
            nxt = i + NBUF

            @pl.when(nxt < NCHUNK)
            def _():
                write(i, b).wait()
                build(nxt, b)
                gather(b).start()

        return carry

    lax.fori_loop(0, NCHUNK // NBUF, lambda k, c: group(k * NBUF, c), 0)

    for b in range(NBUF):
        write(NCHUNK - NBUF + b, b).wait()


_gather = functools.partial(
    pl.kernel,
    out_type=jax.ShapeDtypeStruct((N, D), jnp.float32),
    mesh=plsc.VectorSubcoreMesh(core_axis_name="c", subcore_axis_name="s"),
    scratch_types=[
        pltpu.VMEM((PER_W,), jnp.int32),
        pltpu.VMEM((NBUF, CHUNK), jnp.int32),
        pltpu.VMEM((NBUF, CHUNK, D), jnp.float32),
        pltpu.SemaphoreType.DMA((NBUF,)),
        pltpu.SemaphoreType.DMA((NBUF,)),
    ],
    compiler_params=pltpu.CompilerParams(
        use_tc_tiling_on_sc=False, needs_layout_passes=False),
)(_gather_body)


@jax.jit
def kernel(inputs, table):
    idx = inputs.reshape(NW, PER_W)
    rows = _gather(idx, table)
    out4 = rows.reshape(B // 8, L // 2, 8, 2 * D)
    return out4.transpose(0, 2, 1, 3).reshape(B, L * D)
